# Initial kernel scaffold; baseline (speedup 1.0000x reference)
import jax
import jax.numpy as jnp
from jax.experimental import pallas as pl


def kernel(prediction, ground, trailing_pred, trailing_ground):
    raise NotImplementedError("write your pallas kernel here")


# speedup vs baseline: 94.1774x; 94.1774x over previous
import jax
import jax.numpy as jnp
from jax.experimental import pallas as pl


_N = 16384
_TN = 8 * 16384


def _loss_kernel(p_ref, g_ref, tp_ref, out_ref):
    thresh = jnp.max(tp_ref[...])
    p = p_ref[...]
    g = g_ref[...]
    bce = g * jnp.log(p) + (1.0 - g) * jnp.log(1.0 - p)
    fltr = jnp.logical_and(p > thresh, g == 0.0)
    loss = jnp.where(fltr, bce * 20.0, bce)
    out_ref[...] = (jnp.sum(loss) * (1.0 / _N)).reshape(1, 1)


def kernel(prediction, ground, trailing_pred, trailing_ground):
    p2 = prediction.reshape(128, 128)
    g2 = ground.reshape(128, 128)
    tp2 = trailing_pred.reshape(1024, 128)
    out = pl.pallas_call(
        _loss_kernel,
        out_shape=jax.ShapeDtypeStruct((1, 1), jnp.float32),
    )(p2, g2, tp2)
    return out[0, 0]
